# baseline (speedup 1.0000x reference)
import functools

import jax
import jax.numpy as jnp
from jax import lax
from jax.experimental import pallas as pl
from jax.experimental.pallas import tpu as pltpu
from jax.experimental.pallas import tpu_sc as plsc

N = 10000
NPAD = 10240
F = 128
E = 320000
NC = 2
NS = 16
RPT = NPAD // NS
KP = 96
CHPH = 36
PH = 3
CHP = PH * CHPH
NBUF = 3
PAD_ROW = 10200

RB = 1024
NBLK = NPAD // RB


def _vmesh():
    return plsc.VectorSubcoreMesh(core_axis_name="c", subcore_axis_name="s")


EPT = E // (NC * NS)


def _sc_degree(dst2, zeros_line):
    @functools.partial(
        pl.kernel,
        out_type=jax.ShapeDtypeStruct((NC * NS, NPAD), jnp.float32),
        mesh=_vmesh(),
        compiler_params=pltpu.CompilerParams(needs_layout_passes=False),
        scratch_types=[
            pltpu.VMEM((EPT,), jnp.int32),
            pltpu.VMEM((NPAD,), jnp.float32),
        ],
    )
    def body(dst_hbm, zero_hbm, out_hbm, idx_v, cnt_v):
        c = lax.axis_index("c")
        s = lax.axis_index("s")
        t = c * NS + s
        pltpu.sync_copy(zero_hbm, cnt_v)
        pltpu.sync_copy(dst_hbm.at[t], idx_v)
        ones = jnp.full((16,), 1.0, jnp.float32)

        def step(i, carry):
            v = idx_v[pl.ds(i * 16, 16)]
            plsc.addupdate_scatter(cnt_v, [v], ones)
            return carry

        lax.fori_loop(0, EPT // 16, step, 0)
        pltpu.sync_copy(cnt_v, out_hbm.at[t])

    return body(dst2, zeros_line)


def _sc_gather_scatter(y, src5, dst5, zeros_rows):
    @functools.partial(
        pl.kernel,
        out_type=jax.ShapeDtypeStruct((NC * NPAD, F), jnp.float32),
        mesh=_vmesh(),
        scratch_types=[
            pltpu.VMEM((CHPH, KP), jnp.int32),
            pltpu.VMEM((CHPH, KP), jnp.int32),
            pltpu.VMEM((NBUF, KP, F), jnp.float32),
            pltpu.VMEM_SHARED((NPAD, F), jnp.float32),
            pltpu.SemaphoreType.DMA((NBUF,)),
            pltpu.SemaphoreType.DMA((NBUF,)),
        ],
    )
    def body(y_hbm, src_hbm, dst_hbm, zero_hbm, out_hbm,
             src_v, dst_v, rows_v, acc_s, sem_g, sem_s):
        c = lax.axis_index("c")
        s = lax.axis_index("s")
        t = c * NS + s
        base = c * NPAD + s * RPT
        pltpu.sync_copy(zero_hbm, acc_s.at[pl.ds(s * RPT, RPT)])
        plsc.subcore_barrier()

        def fire_g(j, b):
            pltpu.async_copy(y_hbm.at[src_v.at[j]], rows_v.at[b], sem_g.at[b])

        def wait_g(b):
            pltpu.make_async_copy(y_hbm.at[src_v.at[0]], rows_v.at[b],
                                  sem_g.at[b]).wait()

        def fire_s(j, b):
            pltpu.async_copy(rows_v.at[b], acc_s.at[dst_v.at[j]], sem_s.at[b],
                             add=True)

        def wait_s(b):
            pltpu.make_async_copy(rows_v.at[b], acc_s.at[dst_v.at[0]],
                                  sem_s.at[b]).wait()

        for p in range(PH):
            u = t * PH + p
            pltpu.sync_copy(src_hbm.at[u], src_v)
            pltpu.sync_copy(dst_hbm.at[u], dst_v)

            for b in range(NBUF):
                fire_g(b, b)

            def group(j2, carry):
                for bp in range(NBUF):
                    j = 3 * j2 + bp
                    wait_g(bp)
                    fire_s(j, bp)
                    wait_s(bp)
                    fire_g(j + NBUF, bp)
                return carry

            lax.fori_loop(0, (CHPH - NBUF) // NBUF, group, 0)
            for e in range(NBUF):
                j = CHPH - NBUF + e
                wait_g(e)
                fire_s(j, e)
                wait_s(e)

        plsc.subcore_barrier()
        pltpu.sync_copy(acc_s.at[pl.ds(s * RPT, RPT)], out_hbm.at[pl.ds(base, RPT)])

    return body(y, src5, dst5, zeros_rows)


def _mm_body(x_ref, w_ref, o_ref):
    o_ref[...] = jnp.dot(x_ref[...], w_ref[...],
                         preferred_element_type=jnp.float32)


def _tc_matmul(xp, w):
    m, k1 = xp.shape
    k2 = w.shape[1]
    return pl.pallas_call(
        _mm_body,
        grid=(m // RB,),
        in_specs=[pl.BlockSpec((RB, k1), lambda i: (i, 0)),
                  pl.BlockSpec((k1, k2), lambda i: (0, 0))],
        out_specs=pl.BlockSpec((RB, k2), lambda i: (i, 0)),
        out_shape=jax.ShapeDtypeStruct((m, k2), jnp.float32),
    )(xp, w)


def _dis_body(dp_ref, o_ref):
    ones = jnp.ones((NC * NS, 1), jnp.float32)
    deg = lax.dot_general(dp_ref[...], ones, (((0,), (0,)), ((), ())),
                          preferred_element_type=jnp.float32)
    o_ref[...] = lax.rsqrt(deg + 1.0)


def _tc_dis(degpart):
    return pl.pallas_call(
        _dis_body,
        grid=(NBLK,),
        in_specs=[pl.BlockSpec((NC * NS, RB), lambda i: (0, i))],
        out_specs=pl.BlockSpec((RB, 1), lambda i: (i, 0)),
        out_shape=jax.ShapeDtypeStruct((NPAD, 1), jnp.float32),
    )(degpart)


def _scale_body(xw_ref, dis_ref, y_ref):
    y_ref[...] = xw_ref[...] * dis_ref[...]


def _tc_scale(xw, dis):
    return pl.pallas_call(
        _scale_body,
        grid=(NBLK,),
        in_specs=[pl.BlockSpec((RB, F), lambda i: (i, 0)),
                  pl.BlockSpec((RB, 1), lambda i: (i, 0))],
        out_specs=pl.BlockSpec((RB, F), lambda i: (i, 0)),
        out_shape=jax.ShapeDtypeStruct((NPAD, F), jnp.float32),
    )(xw, dis)


def _mid_body(a0_ref, a1_ref, y_ref, dis_ref, b_ref, w_ref, o_ref):
    d = dis_ref[...]
    h = (a0_ref[...] + a1_ref[...] + y_ref[...]) * d
    h = jnp.maximum(h + b_ref[...], 0.0)
    o_ref[...] = jnp.dot(h, w_ref[...], preferred_element_type=jnp.float32) * d


def _tc_mid(accpart, y, dis, b, w):
    k2 = w.shape[1]
    return pl.pallas_call(
        _mid_body,
        grid=(NBLK,),
        in_specs=[pl.BlockSpec((RB, F), lambda i: (i, 0)),
                  pl.BlockSpec((RB, F), lambda i: (NBLK + i, 0)),
                  pl.BlockSpec((RB, F), lambda i: (i, 0)),
                  pl.BlockSpec((RB, 1), lambda i: (i, 0)),
                  pl.BlockSpec((1, F), lambda i: (0, 0)),
                  pl.BlockSpec((F, k2), lambda i: (0, 0))],
        out_specs=pl.BlockSpec((RB, k2), lambda i: (i, 0)),
        out_shape=jax.ShapeDtypeStruct((NPAD, k2), jnp.float32),
    )(accpart, accpart, y, dis, b, w)


def _fin_body(a0_ref, a1_ref, y_ref, dis_ref, b_ref, w_ref, blin_ref, o_ref):
    h = (a0_ref[...] + a1_ref[...] + y_ref[...]) * dis_ref[...]
    h = jnp.maximum(h + b_ref[...], 0.0)
    o_ref[...] = (jnp.dot(h, w_ref[...], preferred_element_type=jnp.float32)
                  + blin_ref[...])


def _tc_fin(accpart, y, dis, b, wlin, blin):
    k2 = wlin.shape[1]
    return pl.pallas_call(
        _fin_body,
        grid=(NBLK,),
        in_specs=[pl.BlockSpec((RB, F), lambda i: (i, 0)),
                  pl.BlockSpec((RB, F), lambda i: (NBLK + i, 0)),
                  pl.BlockSpec((RB, F), lambda i: (i, 0)),
                  pl.BlockSpec((RB, 1), lambda i: (i, 0)),
                  pl.BlockSpec((1, F), lambda i: (0, 0)),
                  pl.BlockSpec((F, k2), lambda i: (0, 0)),
                  pl.BlockSpec((1, k2), lambda i: (0, 0))],
        out_specs=pl.BlockSpec((RB, k2), lambda i: (i, 0)),
        out_shape=jax.ShapeDtypeStruct((NPAD, k2), jnp.float32),
    )(accpart, accpart, y, dis, b, wlin, blin)


def kernel(x, edge_index, W1, b1, W2, b2, Wlin, blin):
    src = edge_index[0].astype(jnp.int32)
    dst = edge_index[1].astype(jnp.int32)
    srcr = src.reshape(NC * NS, EPT)
    dstr = dst.reshape(NC * NS, EPT)
    pade = CHP * KP - EPT
    src5 = jnp.concatenate(
        [srcr, jnp.zeros((NC * NS, pade), jnp.int32)], axis=1
    ).reshape(NC * NS * PH, CHPH, KP)
    dst5 = jnp.concatenate(
        [dstr, jnp.full((NC * NS, pade), PAD_ROW, jnp.int32)], axis=1
    ).reshape(NC * NS * PH, CHPH, KP)
    dst2 = dstr

    xpad = jnp.concatenate(
        [x, jnp.zeros((NPAD - N, F), jnp.float32)], axis=0)
    zeros_line = jnp.zeros((NPAD,), jnp.float32)
    zeros_rows = jnp.zeros((RPT, F), jnp.float32)

    degpart = _sc_degree(dst2, zeros_line)
    xw1 = _tc_matmul(xpad, W1)
    dis = _tc_dis(degpart)

    y1 = _tc_scale(xw1, dis)
    acc1 = _sc_gather_scatter(y1, src5, dst5, zeros_rows)
    y2 = _tc_mid(acc1, y1, dis, b1.reshape(1, F), W2)

    acc2 = _sc_gather_scatter(y2, src5, dst5, zeros_rows)
    out = _tc_fin(acc2, y2, dis, b2.reshape(1, F), Wlin, blin.reshape(1, -1))
    return out[:N]


# speedup vs baseline: 2.4503x; 2.4503x over previous
import functools

import jax
import jax.numpy as jnp
from jax import lax
from jax.experimental import pallas as pl
from jax.experimental.pallas import tpu as pltpu
from jax.experimental.pallas import tpu_sc as plsc

N = 10000
NPAD = 10240
F = 128
E = 320000
NC = 2
NS = 16
RPT = NPAD // NS
KP = 80
CHPH = 63
PH = 2
CHP = PH * CHPH
NBUF = 3
PAD_ROW = 10200

RB = 1024
NBLK = NPAD // RB


def _vmesh():
    return plsc.VectorSubcoreMesh(core_axis_name="c", subcore_axis_name="s")


EPT = E // (NC * NS)


def _sc_degree(dst2, zeros_line):
    @functools.partial(
        pl.kernel,
        out_type=jax.ShapeDtypeStruct((NC * NS, NPAD), jnp.float32),
        mesh=_vmesh(),
        compiler_params=pltpu.CompilerParams(needs_layout_passes=False),
        scratch_types=[
            pltpu.VMEM((EPT,), jnp.int32),
            pltpu.VMEM((NPAD,), jnp.float32),
        ],
    )
    def body(dst_hbm, zero_hbm, out_hbm, idx_v, cnt_v):
        c = lax.axis_index("c")
        s = lax.axis_index("s")
        t = c * NS + s
        pltpu.sync_copy(zero_hbm, cnt_v)
        pltpu.sync_copy(dst_hbm.at[t], idx_v)
        ones = jnp.full((16,), 1.0, jnp.float32)

        def step(i, carry):
            v = idx_v[pl.ds(i * 16, 16)]
            plsc.addupdate_scatter(cnt_v, [v], ones)
            return carry

        lax.fori_loop(0, EPT // 16, step, 0)
        pltpu.sync_copy(cnt_v, out_hbm.at[t])

    return body(dst2, zeros_line)


def _sc_gather_scatter(y, src5, dst5, zeros_rows):
    @functools.partial(
        pl.kernel,
        out_type=jax.ShapeDtypeStruct((NC * NPAD, F), jnp.float32),
        mesh=_vmesh(),
        scratch_types=[
            pltpu.VMEM((CHPH, KP), jnp.int32),
            pltpu.VMEM((CHPH, KP), jnp.int32),
            pltpu.VMEM((NBUF, KP, F), jnp.float32),
            pltpu.VMEM_SHARED((NPAD, F), jnp.float32),
            pltpu.SemaphoreType.DMA((NBUF,)),
            pltpu.SemaphoreType.DMA((NBUF,)),
        ],
    )
    def body(y_hbm, src_hbm, dst_hbm, zero_hbm, out_hbm,
             src_v, dst_v, rows_v, acc_s, sem_g, sem_s):
        c = lax.axis_index("c")
        s = lax.axis_index("s")
        t = c * NS + s
        base = c * NPAD + s * RPT
        pltpu.sync_copy(zero_hbm, acc_s.at[pl.ds(s * RPT, RPT)])
        plsc.subcore_barrier()

        def fire_g(j, b):
            pltpu.async_copy(y_hbm.at[src_v.at[j]], rows_v.at[b], sem_g.at[b])

        def wait_g(b):
            pltpu.make_async_copy(y_hbm.at[src_v.at[0]], rows_v.at[b],
                                  sem_g.at[b]).wait()

        def fire_s(j, b):
            pltpu.async_copy(rows_v.at[b], acc_s.at[dst_v.at[j]], sem_s.at[b],
                             add=True)

        def wait_s(b):
            pltpu.make_async_copy(rows_v.at[b], acc_s.at[dst_v.at[0]],
                                  sem_s.at[b]).wait()

        for p in range(PH):
            u = t * PH + p
            pltpu.sync_copy(src_hbm.at[u], src_v)
            pltpu.sync_copy(dst_hbm.at[u], dst_v)

            for b in range(NBUF):
                fire_g(b, b)

            def group(j2, carry):
                for bp in range(NBUF):
                    j = 3 * j2 + bp
                    wait_g(bp)
                    fire_s(j, bp)
                    wait_s(bp)
                    fire_g(j + NBUF, bp)
                return carry

            lax.fori_loop(0, (CHPH - NBUF) // NBUF, group, 0)
            for e in range(NBUF):
                j = CHPH - NBUF + e
                wait_g(e)
                fire_s(j, e)
                wait_s(e)

        plsc.subcore_barrier()
        pltpu.sync_copy(acc_s.at[pl.ds(s * RPT, RPT)], out_hbm.at[pl.ds(base, RPT)])

    return body(y, src5, dst5, zeros_rows)


def _mm_body(x_ref, w_ref, o_ref):
    o_ref[...] = jnp.dot(x_ref[...], w_ref[...],
                         preferred_element_type=jnp.float32)


def _tc_matmul(xp, w):
    m, k1 = xp.shape
    k2 = w.shape[1]
    return pl.pallas_call(
        _mm_body,
        grid=(m // RB,),
        in_specs=[pl.BlockSpec((RB, k1), lambda i: (i, 0)),
                  pl.BlockSpec((k1, k2), lambda i: (0, 0))],
        out_specs=pl.BlockSpec((RB, k2), lambda i: (i, 0)),
        out_shape=jax.ShapeDtypeStruct((m, k2), jnp.float32),
    )(xp, w)


def _dis_scale_body(dp_ref, xw_ref, dis_ref, y_ref):
    ones = jnp.ones((NC * NS, 1), jnp.float32)
    deg = lax.dot_general(dp_ref[...], ones, (((0,), (0,)), ((), ())),
                          preferred_element_type=jnp.float32)
    d = lax.rsqrt(deg + 1.0)
    dis_ref[...] = d
    y_ref[...] = xw_ref[...] * d


def _tc_dis_scale(degpart, xw):
    return pl.pallas_call(
        _dis_scale_body,
        grid=(NBLK,),
        in_specs=[pl.BlockSpec((NC * NS, RB), lambda i: (0, i)),
                  pl.BlockSpec((RB, F), lambda i: (i, 0))],
        out_specs=[pl.BlockSpec((RB, 1), lambda i: (i, 0)),
                   pl.BlockSpec((RB, F), lambda i: (i, 0))],
        out_shape=[jax.ShapeDtypeStruct((NPAD, 1), jnp.float32),
                   jax.ShapeDtypeStruct((NPAD, F), jnp.float32)],
    )(degpart, xw)


def _mid_body(a0_ref, a1_ref, y_ref, dis_ref, b_ref, w_ref, o_ref):
    d = dis_ref[...]
    h = (a0_ref[...] + a1_ref[...] + y_ref[...]) * d
    h = jnp.maximum(h + b_ref[...], 0.0)
    o_ref[...] = jnp.dot(h, w_ref[...], preferred_element_type=jnp.float32) * d


def _tc_mid(accpart, y, dis, b, w):
    k2 = w.shape[1]
    return pl.pallas_call(
        _mid_body,
        grid=(NBLK,),
        in_specs=[pl.BlockSpec((RB, F), lambda i: (i, 0)),
                  pl.BlockSpec((RB, F), lambda i: (NBLK + i, 0)),
                  pl.BlockSpec((RB, F), lambda i: (i, 0)),
                  pl.BlockSpec((RB, 1), lambda i: (i, 0)),
                  pl.BlockSpec((1, F), lambda i: (0, 0)),
                  pl.BlockSpec((F, k2), lambda i: (0, 0))],
        out_specs=pl.BlockSpec((RB, k2), lambda i: (i, 0)),
        out_shape=jax.ShapeDtypeStruct((NPAD, k2), jnp.float32),
    )(accpart, accpart, y, dis, b, w)


def _fin_body(a0_ref, a1_ref, y_ref, dis_ref, b_ref, w_ref, blin_ref, o_ref):
    h = (a0_ref[...] + a1_ref[...] + y_ref[...]) * dis_ref[...]
    h = jnp.maximum(h + b_ref[...], 0.0)
    o_ref[...] = (jnp.dot(h, w_ref[...], preferred_element_type=jnp.float32)
                  + blin_ref[...])


def _tc_fin(accpart, y, dis, b, wlin, blin):
    k2 = wlin.shape[1]
    return pl.pallas_call(
        _fin_body,
        grid=(NBLK,),
        in_specs=[pl.BlockSpec((RB, F), lambda i: (i, 0)),
                  pl.BlockSpec((RB, F), lambda i: (NBLK + i, 0)),
                  pl.BlockSpec((RB, F), lambda i: (i, 0)),
                  pl.BlockSpec((RB, 1), lambda i: (i, 0)),
                  pl.BlockSpec((1, F), lambda i: (0, 0)),
                  pl.BlockSpec((F, k2), lambda i: (0, 0)),
                  pl.BlockSpec((1, k2), lambda i: (0, 0))],
        out_specs=pl.BlockSpec((RB, k2), lambda i: (i, 0)),
        out_shape=jax.ShapeDtypeStruct((NPAD, k2), jnp.float32),
    )(accpart, accpart, y, dis, b, wlin, blin)


def kernel(x, edge_index, W1, b1, W2, b2, Wlin, blin):
    src = edge_index[0].astype(jnp.int32)
    dst = edge_index[1].astype(jnp.int32)
    srcr = src.reshape(NC * NS, EPT)
    dstr = dst.reshape(NC * NS, EPT)
    pade = CHP * KP - EPT
    src5 = jnp.concatenate(
        [srcr, jnp.zeros((NC * NS, pade), jnp.int32)], axis=1
    ).reshape(NC * NS * PH, CHPH, KP)
    dst5 = jnp.concatenate(
        [dstr, jnp.full((NC * NS, pade), PAD_ROW, jnp.int32)], axis=1
    ).reshape(NC * NS * PH, CHPH, KP)
    dst2 = dstr

    xpad = jnp.concatenate(
        [x, jnp.zeros((NPAD - N, F), jnp.float32)], axis=0)
    zeros_line = jnp.zeros((NPAD,), jnp.float32)
    zeros_rows = jnp.zeros((RPT, F), jnp.float32)

    degpart = _sc_degree(dst2, zeros_line)
    xw1 = _tc_matmul(xpad, W1)
    dis, y1 = _tc_dis_scale(degpart, xw1)

    acc1 = _sc_gather_scatter(y1, src5, dst5, zeros_rows)
    y2 = _tc_mid(acc1, y1, dis, b1.reshape(1, F), W2)

    acc2 = _sc_gather_scatter(y2, src5, dst5, zeros_rows)
    out = _tc_fin(acc2, y2, dis, b2.reshape(1, F), Wlin, blin.reshape(1, -1))
    return out[:N]
